# baseline (speedup 1.0000x reference)
import functools

import jax
import jax.numpy as jnp
from jax import lax
from jax.experimental import pallas as pl
from jax.experimental.pallas import tpu as pltpu
from jax.experimental.pallas import tpu_sc as plsc

_B = 4096
_L = 200
_DIM = 64
_NTOT = _B * _L
_NC = 2
_NS = 16
_NW = _NC * _NS
_PER_W = _NTOT // _NW
_B_PER_W = _B // _NW
_CHUNK = _L
_NCHUNK = _B_PER_W


@functools.partial(
    pl.kernel,
    mesh=plsc.VectorSubcoreMesh(core_axis_name="c", subcore_axis_name="s"),
    out_type=jax.ShapeDtypeStruct((_B, _L, _DIM), jnp.float32),
    scratch_types=[
        pltpu.VMEM((_PER_W,), jnp.int32),
        pltpu.VMEM((_CHUNK, _DIM), jnp.float32),
        pltpu.VMEM((_CHUNK, _DIM), jnp.float32),
        pltpu.SemaphoreType.DMA,
        pltpu.SemaphoreType.DMA,
        pltpu.SemaphoreType.DMA,
        pltpu.SemaphoreType.DMA,
    ],
    compiler_params=pltpu.CompilerParams(use_tc_tiling_on_sc=False),
)
def _lookup(idx_hbm, table_hbm, out_hbm, idx_v, rows0, rows1, sg0, sg1, so0, so1):
    wid = lax.axis_index("s") * _NC + lax.axis_index("c")
    b0 = wid * _B_PER_W
    rows = (rows0, rows1)
    sg = (sg0, sg1)
    so = (so0, so1)

    def fire_gather(c, b):
        pltpu.async_copy(
            table_hbm.at[idx_v.at[pl.ds(c * _CHUNK, _CHUNK)]],
            rows[b],
            sg[b],
        )

    def wait_gather(b):
        pltpu.make_async_copy(
            table_hbm.at[pl.ds(0, _CHUNK)], rows[b], sg[b]
        ).wait()

    pltpu.sync_copy(idx_hbm.at[pl.ds(wid * _PER_W, _PER_W)], idx_v)

    fire_gather(0, 0)
    fire_gather(1, 1)

    def body(g, carry):
        for b in range(2):
            c = 2 * g + b
            wait_gather(b)
            w = pltpu.async_copy(rows[b], out_hbm.at[b0 + c], so[b])
            w.wait()
            fire_gather(c + 2, b)
        return carry

    lax.fori_loop(0, _NCHUNK // 2 - 1, body, 0)

    for b in range(2):
        c = _NCHUNK - 2 + b
        wait_gather(b)
        pltpu.async_copy(rows[b], out_hbm.at[b0 + c], so[b]).wait()


def kernel(idxes, pe):
    idx_flat = idxes.astype(jnp.int32).reshape(_NTOT)
    return _lookup(idx_flat, pe)


# speedup vs baseline: 1.1002x; 1.1002x over previous
import functools

import jax
import jax.numpy as jnp
from jax import lax
from jax.experimental import pallas as pl
from jax.experimental.pallas import tpu as pltpu
from jax.experimental.pallas import tpu_sc as plsc

_B = 4096
_L = 200
_DIM = 64
_NUM_EMB = 8192
_NTOT = _B * _L
_NC = 2
_NS = 16
_NW = _NC * _NS
_PER_W = _NTOT // _NW
_B_PER_W = _B // _NW
_CHUNK = _L
_NCHUNK = _B_PER_W


@functools.partial(
    pl.kernel,
    mesh=plsc.VectorSubcoreMesh(core_axis_name="c", subcore_axis_name="s"),
    out_type=jax.ShapeDtypeStruct((_B, _L, _DIM), jnp.float32),
    scratch_types=[
        pltpu.VMEM((_PER_W,), jnp.int32),
        pltpu.VMEM((_CHUNK, 128), jnp.float32),
        pltpu.VMEM((_CHUNK, 128), jnp.float32),
        pltpu.VMEM((_CHUNK, _DIM), jnp.float32),
        pltpu.VMEM((_CHUNK, _DIM), jnp.float32),
        pltpu.SemaphoreType.DMA,
        pltpu.SemaphoreType.DMA,
        pltpu.SemaphoreType.DMA,
        pltpu.SemaphoreType.DMA,
    ],
    compiler_params=pltpu.CompilerParams(use_tc_tiling_on_sc=True),
)
def _lookup(idx_hbm, table_hbm, out_hbm, idx_v, g0, g1, p0, p1,
            sg0, sg1, so0, so1):
    wid = lax.axis_index("s") * _NC + lax.axis_index("c")
    b0 = wid * _B_PER_W
    gath = (g0, g1)
    pack = (p0, p1)
    sg = (sg0, sg1)
    so = (so0, so1)

    def fire_gather(c, b):
        pltpu.async_copy(
            table_hbm.at[idx_v.at[pl.ds(c * _CHUNK, _CHUNK)]],
            gath[b],
            sg[b],
        )

    def wait_gather(b):
        pltpu.make_async_copy(
            table_hbm.at[pl.ds(0, _CHUNK)], gath[b], sg[b]
        ).wait()

    def repack(b):
        def rbody(i, carry):
            for rr in range(4):
                r = 4 * i + rr
                for j in range(_DIM // 16):
                    pack[b][r, pl.ds(j * 16, 16)] = gath[b][r, pl.ds(j * 16, 16)]
            return carry
        lax.fori_loop(0, _CHUNK // 4, rbody, 0)

    pltpu.sync_copy(idx_hbm.at[pl.ds(wid * _PER_W, _PER_W)], idx_v)

    fire_gather(0, 0)
    fire_gather(1, 1)

    def body(g, carry):
        for b in range(2):
            c = 2 * g + b
            wait_gather(b)
            repack(b)
            w = pltpu.async_copy(pack[b], out_hbm.at[b0 + c], so[b])
            fire_gather(c + 2, b)
            w.wait()
        return carry

    lax.fori_loop(0, _NCHUNK // 2 - 1, body, 0)

    for b in range(2):
        c = _NCHUNK - 2 + b
        wait_gather(b)
        repack(b)
        pltpu.async_copy(pack[b], out_hbm.at[b0 + c], so[b]).wait()


def kernel(idxes, pe):
    idx_flat = idxes.astype(jnp.int32).reshape(_NTOT)
    pe128 = jnp.concatenate(
        [pe, jnp.zeros((_NUM_EMB, 128 - _DIM), jnp.float32)], axis=1
    )
    return _lookup(idx_flat, pe128)


# speedup vs baseline: 1.3074x; 1.1884x over previous
import functools

import jax
import jax.numpy as jnp
from jax import lax
from jax.experimental import pallas as pl
from jax.experimental.pallas import tpu as pltpu
from jax.experimental.pallas import tpu_sc as plsc

_B = 4096
_L = 200
_DIM = 64
_NUM_EMB = 8192
_NTOT = _B * _L
_NC = 2
_NS = 16
_NW = _NC * _NS
_R = _DIM // 8
_Q = _NW // _R
_LQ = _L // _Q
_TBLW = 8 * _NUM_EMB


@functools.partial(
    pl.kernel,
    mesh=plsc.VectorSubcoreMesh(core_axis_name="c", subcore_axis_name="s"),
    out_type=jax.ShapeDtypeStruct((_L, 8, 32, 8, 128), jnp.float32),
    scratch_types=[
        pltpu.VMEM((_TBLW,), jnp.float32),
        pltpu.VMEM((_B,), jnp.int32),
        pltpu.VMEM((_B,), jnp.int32),
        pltpu.VMEM((16, 8, 128), jnp.float32),
        pltpu.VMEM((16, 8, 128), jnp.float32),
        pltpu.SemaphoreType.DMA,
        pltpu.SemaphoreType.DMA,
        pltpu.SemaphoreType.DMA,
        pltpu.SemaphoreType.DMA,
    ],
    compiler_params=pltpu.CompilerParams(
        use_tc_tiling_on_sc=True, needs_layout_passes=False
    ),
)
def _lookup(idx_hbm, table_hbm, out_hbm, tbl, ib0, ib1, ob0, ob1,
            si0, si1, so0, so1):
    wid = lax.axis_index("s") * _NC + lax.axis_index("c")
    r = wid % _R
    l0 = (wid // _R) * _LQ
    ib = (ib0, ib1)
    ob = (ob0, ob1)
    si = (si0, si1)
    so = (so0, so1)

    def fire_idx(li, p):
        pltpu.async_copy(idx_hbm.at[pl.ds((l0 + li) * _B, _B)], ib[p], si[p])

    def wait_idx(p):
        pltpu.make_async_copy(idx_hbm.at[pl.ds(0, _B)], ib[p], si[p]).wait()

    def build(ibuf, obuf, h):
        def cbody(c, carry):
            for gg in range(8):
                idxv = ibuf[pl.ds(h * 2048 + c * 128 + gg * 16, 16)]
                for d in range(8):
                    v = plsc.load_gather(tbl, [idxv + d * _NUM_EMB])
                    obuf[c, d, pl.ds(gg * 16, 16)] = v
            return carry
        lax.fori_loop(0, 16, cbody, 0)

    def fire_out(li, hb):
        pltpu.async_copy(
            ob[hb], out_hbm.at[l0 + li, r, pl.ds(16 * hb, 16)], so[hb]
        )

    def wait_out(hb):
        pltpu.make_async_copy(
            ob[hb], out_hbm.at[0, r, pl.ds(16 * hb, 16)], so[hb]
        ).wait()

    pltpu.sync_copy(table_hbm.at[pl.ds(r * _TBLW, _TBLW)], tbl)
    fire_idx(0, 0)
    fire_idx(1, 1)

    def do_l(li, p, first, prefetch):
        wait_idx(p)
        if prefetch:
            fire_idx(li + 1, 1 - p)
        for hb in range(2):
            if not first:
                wait_out(hb)
            build(ib[p], ob[hb], hb)
            fire_out(li, hb)

    do_l(0, 0, True, False)

    def body(g2, carry):
        do_l(1 + 2 * g2, 1, False, True)
        do_l(2 + 2 * g2, 0, False, True)
        return carry

    lax.fori_loop(0, (_LQ - 2) // 2, body, 0)

    do_l(_LQ - 1, 1, False, False)
    wait_out(0)
    wait_out(1)


def kernel(idxes, pe):
    idx_t = idxes.astype(jnp.int32).T.reshape(_NTOT)
    pe_t = pe.T.reshape(_NUM_EMB * _DIM)
    out = _lookup(idx_t, pe_t)
    return jnp.transpose(out, (2, 4, 0, 1, 3)).reshape(_B, _L, _DIM)


# speedup vs baseline: 3.0585x; 2.3393x over previous
import functools

import jax
import jax.numpy as jnp
from jax import lax
from jax.experimental import pallas as pl
from jax.experimental.pallas import tpu as pltpu
from jax.experimental.pallas import tpu_sc as plsc

_B = 4096
_L = 200
_DIM = 64
_NUM_EMB = 8192
_NTOT = _B * _L
_NC = 2
_NS = 16
_NW = _NC * _NS
_R = _DIM // 8
_Q = _NW // _R
_LQ = _L // _Q
_TBLW = 8 * _NUM_EMB


@functools.partial(
    pl.kernel,
    mesh=plsc.VectorSubcoreMesh(core_axis_name="c", subcore_axis_name="s"),
    out_type=jax.ShapeDtypeStruct((_L, 8, 32, 8, 128), jnp.float32),
    scratch_types=[
        pltpu.VMEM((_TBLW,), jnp.float32),
        pltpu.VMEM((_B,), jnp.int32),
        pltpu.VMEM((_B,), jnp.int32),
        pltpu.VMEM((16, 8, 128), jnp.float32),
        pltpu.VMEM((16, 8, 128), jnp.float32),
        pltpu.SemaphoreType.DMA,
        pltpu.SemaphoreType.DMA,
        pltpu.SemaphoreType.DMA,
        pltpu.SemaphoreType.DMA,
    ],
    compiler_params=pltpu.CompilerParams(
        use_tc_tiling_on_sc=True, needs_layout_passes=False
    ),
)
def _lookup(idx_hbm, table_hbm, out_hbm, tbl, ib0, ib1, ob0, ob1,
            si0, si1, so0, so1):
    wid = lax.axis_index("s") * _NC + lax.axis_index("c")
    r = wid % _R
    l0 = (wid // _R) * _LQ
    ib = (ib0, ib1)
    ob = (ob0, ob1)
    si = (si0, si1)
    so = (so0, so1)

    def fire_idx(li, p):
        pltpu.async_copy(idx_hbm.at[pl.ds((l0 + li) * _B, _B)], ib[p], si[p])

    def wait_idx(p):
        pltpu.make_async_copy(idx_hbm.at[pl.ds(0, _B)], ib[p], si[p]).wait()

    def build(ibuf, obuf, h):
        @plsc.parallel_loop(0, 16, unroll=2)
        def cbody(c):
            for gg in range(8):
                idxv = ibuf[pl.ds(h * 2048 + c * 128 + gg * 16, 16)]
                for d in range(8):
                    v = plsc.load_gather(tbl, [idxv + d * _NUM_EMB])
                    obuf[c, d, pl.ds(gg * 16, 16)] = v

    def fire_out(li, hb):
        pltpu.async_copy(
            ob[hb], out_hbm.at[l0 + li, r, pl.ds(16 * hb, 16)], so[hb]
        )

    def wait_out(hb):
        pltpu.make_async_copy(
            ob[hb], out_hbm.at[0, r, pl.ds(16 * hb, 16)], so[hb]
        ).wait()

    pltpu.sync_copy(table_hbm.at[pl.ds(r * _TBLW, _TBLW)], tbl)
    fire_idx(0, 0)
    fire_idx(1, 1)

    def do_l(li, p, first, prefetch):
        wait_idx(p)
        if prefetch:
            fire_idx(li + 1, 1 - p)
        for hb in range(2):
            if not first:
                wait_out(hb)
            build(ib[p], ob[hb], hb)
            fire_out(li, hb)

    do_l(0, 0, True, False)

    def body(g2, carry):
        do_l(1 + 2 * g2, 1, False, True)
        do_l(2 + 2 * g2, 0, False, True)
        return carry

    lax.fori_loop(0, (_LQ - 2) // 2, body, 0)

    do_l(_LQ - 1, 1, False, False)
    wait_out(0)
    wait_out(1)


def kernel(idxes, pe):
    idx_t = idxes.astype(jnp.int32).T.reshape(_NTOT)
    pe_t = pe.T.reshape(_NUM_EMB * _DIM)
    out = _lookup(idx_t, pe_t)
    return jnp.transpose(out, (2, 4, 0, 1, 3)).reshape(_B, _L, _DIM)
